# baseline (speedup 1.0000x reference)
import functools
import jax
import jax.numpy as jnp
from jax import lax
from jax.experimental import pallas as pl
from jax.experimental.pallas import tpu as pltpu
from jax.experimental.pallas import tpu_sc as plsc

_FMAX = 200
_CHUNKS = [200] * 5


def _sc_mel_body(melt_h, melp_h, post_h, mmask_h, out_h,
                 bt, bp, bo, mbuf, obuf, sems):
    T_mel = 1000
    n_mels = 80
    nv = n_mels // 16
    wid = lax.axis_index("s") * 2 + lax.axis_index("c")
    b = wid
    spb = T_mel * n_mels
    offs = [sum(_CHUNKS[:i]) for i in range(len(_CHUNKS))]

    pltpu.sync_copy(mmask_h.at[pl.ds(b * T_mel, T_mel)],
                    mbuf.at[pl.ds(0, T_mel)])

    def start(c, slot):
        e0, esz = b * spb + offs[c] * n_mels, _CHUNKS[c] * n_mels
        return [
            pltpu.async_copy(melt_h.at[pl.ds(e0, esz)],
                             bt.at[slot, pl.ds(0, esz)], sems.at[slot, 0]),
            pltpu.async_copy(melp_h.at[pl.ds(e0, esz)],
                             bp.at[slot, pl.ds(0, esz)], sems.at[slot, 1]),
            pltpu.async_copy(post_h.at[pl.ds(e0, esz)],
                             bo.at[slot, pl.ds(0, esz)], sems.at[slot, 2]),
        ]

    pending = {0: start(0, 0)}

    zero = jnp.zeros((16,), jnp.float32)
    accs = (zero, zero, zero)

    for c in range(len(_CHUNKS)):
        slot = c % 2
        if c + 1 < len(_CHUNKS):
            pending[c + 1] = start(c + 1, 1 - slot)
        for h in pending.pop(c):
            h.wait()

        def frame_body(i, carry, slot=slot, c=c):
            accp, accq, accm = carry
            mv16 = mbuf[pl.ds(offs[c] + i, 16)]
            mv = jnp.full((16,), mv16[0], jnp.float32)
            base = i * n_mels
            sp = None
            sq = None
            for k in range(nv):
                tv = bt[slot, pl.ds(base + k * 16, 16)]
                dp = jnp.abs(bp[slot, pl.ds(base + k * 16, 16)] - tv)
                dq = jnp.abs(bo[slot, pl.ds(base + k * 16, 16)] - tv)
                sp = dp if sp is None else sp + dp
                sq = dq if sq is None else sq + dq
            return (accp + mv * sp, accq + mv * sq, accm + mv)

        accs = lax.fori_loop(0, _CHUNKS[c], frame_body, accs, unroll=8)

    obuf[pl.ds(0, 16)] = accs[0]
    obuf[pl.ds(16, 16)] = accs[1]
    obuf[pl.ds(32, 16)] = accs[2]
    pltpu.sync_copy(obuf, out_h.at[wid])


def _sc_mel_sums(melt, melp, post, mmask_flat):
    B, T_mel, n_mels = 32, 1000, 80
    mesh = plsc.VectorSubcoreMesh(core_axis_name="c", subcore_axis_name="s")
    kfn = pl.kernel(
        _sc_mel_body,
        out_type=jax.ShapeDtypeStruct((B, 128), jnp.float32),
        mesh=mesh,
        scratch_types=[
            pltpu.VMEM((2, _FMAX * n_mels), jnp.float32),
            pltpu.VMEM((2, _FMAX * n_mels), jnp.float32),
            pltpu.VMEM((2, _FMAX * n_mels), jnp.float32),
            pltpu.VMEM((T_mel + 24,), jnp.float32),
            pltpu.VMEM((128,), jnp.float32),
            pltpu.SemaphoreType.DMA((2, 3)),
        ],
    )
    return kfn(melt, melp, post, mmask_flat)


def _tc_text_body(pt_ref, pp_ref, et_ref, ep_ref, ldp_ref, dur_ref, tm_ref,
                  out_ref):
    tm = tm_ref[...]
    pe = (pp_ref[...] - pt_ref[...]) ** 2
    ee = (ep_ref[...] - et_ref[...]) ** 2
    ldt = jnp.log(dur_ref[...] + 1.0)
    de = (ldp_ref[...] - ldt) ** 2
    out_ref[0] = jnp.sum(pe * tm)
    out_ref[1] = jnp.sum(ee * tm)
    out_ref[2] = jnp.sum(de * tm)
    out_ref[3] = jnp.sum(tm)


def kernel(mel_targets, pitch_targets, energy_targets, duration_targets,
           mel_predictions, postnet_mel_predictions, pitch_predictions,
           energy_predictions, log_duration_predictions, text_masks, mel_masks):
    B, T_mel, n_mels = mel_targets.shape

    tm = jnp.logical_not(text_masks).astype(jnp.float32)
    mm_flat = jnp.logical_not(mel_masks).astype(jnp.float32).reshape(B * T_mel)
    dur_f = duration_targets.astype(jnp.float32)

    parts = _sc_mel_sums(mel_targets.reshape(-1), mel_predictions.reshape(-1),
                         postnet_mel_predictions.reshape(-1), mm_flat)

    tsums = pl.pallas_call(
        _tc_text_body,
        out_specs=pl.BlockSpec(memory_space=pltpu.SMEM),
        out_shape=jax.ShapeDtypeStruct((4,), jnp.float32),
    )(pitch_targets, pitch_predictions, energy_targets, energy_predictions,
      log_duration_predictions, dur_f, tm)

    mel_num = jnp.sum(parts[:, 0:16])
    post_num = jnp.sum(parts[:, 16:32])
    mel_msum = jnp.sum(parts[:, 32:48]) / 16.0

    n_mels_f = jnp.float32(n_mels)
    mel_loss = mel_num / (mel_msum * n_mels_f)
    postnet_mel_loss = post_num / (mel_msum * n_mels_f)
    pitch_loss = tsums[0] / tsums[3]
    energy_loss = tsums[1] / tsums[3]
    duration_loss = tsums[2] / tsums[3]
    total_loss = (mel_loss + postnet_mel_loss + duration_loss
                  + pitch_loss + energy_loss)
    return (total_loss, mel_loss, postnet_mel_loss, pitch_loss,
            energy_loss, duration_loss)


# speedup vs baseline: 6.6503x; 6.6503x over previous
import jax
import jax.numpy as jnp
from jax.experimental import pallas as pl
from jax.experimental.pallas import tpu as pltpu


def _loss_body(melt_ref, melp_ref, post_ref, mmask_ref,
               pt_ref, pp_ref, et_ref, ep_ref, ldp_ref, dur_ref, tm_ref,
               out_ref):
    step = pl.program_id(0)

    @pl.when(step == 0)
    def _():
        tm = tm_ref[...]
        pe = (pp_ref[...] - pt_ref[...]) ** 2
        ee = (ep_ref[...] - et_ref[...]) ** 2
        ldt = jnp.log(dur_ref[...] + 1.0)
        de = (ldp_ref[...] - ldt) ** 2
        out_ref[0] = 0.0
        out_ref[1] = 0.0
        out_ref[2] = 0.0
        out_ref[3] = jnp.sum(pe * tm)
        out_ref[4] = jnp.sum(ee * tm)
        out_ref[5] = jnp.sum(de * tm)
        out_ref[6] = jnp.sum(tm)
        out_ref[7] = 0.0

    t = melt_ref[...]
    m = mmask_ref[...]
    d1 = jnp.abs(melp_ref[...] - t) * m
    d2 = jnp.abs(post_ref[...] - t) * m
    out_ref[0] += jnp.sum(d1)
    out_ref[1] += jnp.sum(d2)
    out_ref[2] += jnp.sum(m)


def kernel(mel_targets, pitch_targets, energy_targets, duration_targets,
           mel_predictions, postnet_mel_predictions, pitch_predictions,
           energy_predictions, log_duration_predictions, text_masks, mel_masks):
    B, T_mel, n_mels = mel_targets.shape
    T_text = pitch_targets.shape[1]

    tm = jnp.logical_not(text_masks).astype(jnp.float32)
    mm = jnp.logical_not(mel_masks).astype(jnp.float32).reshape(B, 1, T_mel)
    dur_f = duration_targets.astype(jnp.float32)

    mt = jnp.swapaxes(mel_targets, 1, 2)
    mp = jnp.swapaxes(mel_predictions, 1, 2)
    po = jnp.swapaxes(postnet_mel_predictions, 1, 2)

    BB = 4
    sums = pl.pallas_call(
        _loss_body,
        grid=(B // BB,),
        in_specs=[
            pl.BlockSpec((BB, n_mels, T_mel), lambda b: (b, 0, 0)),
            pl.BlockSpec((BB, n_mels, T_mel), lambda b: (b, 0, 0)),
            pl.BlockSpec((BB, n_mels, T_mel), lambda b: (b, 0, 0)),
            pl.BlockSpec((BB, 1, T_mel), lambda b: (b, 0, 0)),
            pl.BlockSpec((B, T_text), lambda b: (0, 0)),
            pl.BlockSpec((B, T_text), lambda b: (0, 0)),
            pl.BlockSpec((B, T_text), lambda b: (0, 0)),
            pl.BlockSpec((B, T_text), lambda b: (0, 0)),
            pl.BlockSpec((B, T_text), lambda b: (0, 0)),
            pl.BlockSpec((B, T_text), lambda b: (0, 0)),
            pl.BlockSpec((B, T_text), lambda b: (0, 0)),
        ],
        out_specs=pl.BlockSpec(memory_space=pltpu.SMEM),
        out_shape=jax.ShapeDtypeStruct((8,), jnp.float32),
    )(mt, mp, po, mm,
      pitch_targets, pitch_predictions, energy_targets, energy_predictions,
      log_duration_predictions, dur_f, tm)

    n_mels_f = jnp.float32(n_mels)
    mel_loss = sums[0] / (sums[2] * n_mels_f)
    postnet_mel_loss = sums[1] / (sums[2] * n_mels_f)
    pitch_loss = sums[3] / sums[6]
    energy_loss = sums[4] / sums[6]
    duration_loss = sums[5] / sums[6]
    total_loss = (mel_loss + postnet_mel_loss + duration_loss
                  + pitch_loss + energy_loss)
    return (total_loss, mel_loss, postnet_mel_loss, pitch_loss,
            energy_loss, duration_loss)


# speedup vs baseline: 7.2988x; 1.0975x over previous
import jax
import jax.numpy as jnp
from jax.experimental import pallas as pl
from jax.experimental.pallas import tpu as pltpu


def _loss_body(melt_ref, melp_ref, post_ref, mmask_ref,
               pt_ref, pp_ref, et_ref, ep_ref, ldp_ref, dur_ref, tm_ref,
               out_ref):
    step = pl.program_id(0)

    @pl.when(step == 0)
    def _():
        tm = jnp.where(tm_ref[...], 0.0, 1.0)
        pe = (pp_ref[...] - pt_ref[...]) ** 2
        ee = (ep_ref[...] - et_ref[...]) ** 2
        ldt = jnp.log(dur_ref[...].astype(jnp.float32) + 1.0)
        de = (ldp_ref[...] - ldt) ** 2
        out_ref[0] = 0.0
        out_ref[1] = 0.0
        out_ref[2] = 0.0
        out_ref[3] = jnp.sum(pe * tm)
        out_ref[4] = jnp.sum(ee * tm)
        out_ref[5] = jnp.sum(de * tm)
        out_ref[6] = jnp.sum(tm)
        out_ref[7] = 0.0

    t = melt_ref[...]
    m = jnp.where(mmask_ref[...], 0.0, 1.0)
    mb = m[:, None, :]
    d1 = jnp.abs(melp_ref[...] - t) * mb
    d2 = jnp.abs(post_ref[...] - t) * mb
    out_ref[0] += jnp.sum(d1)
    out_ref[1] += jnp.sum(d2)
    out_ref[2] += jnp.sum(m)


def kernel(mel_targets, pitch_targets, energy_targets, duration_targets,
           mel_predictions, postnet_mel_predictions, pitch_predictions,
           energy_predictions, log_duration_predictions, text_masks, mel_masks):
    B, T_mel, n_mels = mel_targets.shape
    T_text = pitch_targets.shape[1]

    mt = jnp.swapaxes(mel_targets, 1, 2)
    mp = jnp.swapaxes(mel_predictions, 1, 2)
    po = jnp.swapaxes(postnet_mel_predictions, 1, 2)

    BB = 8
    sums = pl.pallas_call(
        _loss_body,
        grid=(B // BB,),
        in_specs=[
            pl.BlockSpec((BB, n_mels, T_mel), lambda b: (b, 0, 0)),
            pl.BlockSpec((BB, n_mels, T_mel), lambda b: (b, 0, 0)),
            pl.BlockSpec((BB, n_mels, T_mel), lambda b: (b, 0, 0)),
            pl.BlockSpec((BB, T_mel), lambda b: (b, 0)),
            pl.BlockSpec((B, T_text), lambda b: (0, 0)),
            pl.BlockSpec((B, T_text), lambda b: (0, 0)),
            pl.BlockSpec((B, T_text), lambda b: (0, 0)),
            pl.BlockSpec((B, T_text), lambda b: (0, 0)),
            pl.BlockSpec((B, T_text), lambda b: (0, 0)),
            pl.BlockSpec((B, T_text), lambda b: (0, 0)),
            pl.BlockSpec((B, T_text), lambda b: (0, 0)),
        ],
        out_specs=pl.BlockSpec(memory_space=pltpu.SMEM),
        out_shape=jax.ShapeDtypeStruct((8,), jnp.float32),
    )(mt, mp, po, mel_masks,
      pitch_targets, pitch_predictions, energy_targets, energy_predictions,
      log_duration_predictions, duration_targets, text_masks)

    n_mels_f = jnp.float32(n_mels)
    mel_loss = sums[0] / (sums[2] * n_mels_f)
    postnet_mel_loss = sums[1] / (sums[2] * n_mels_f)
    pitch_loss = sums[3] / sums[6]
    energy_loss = sums[4] / sums[6]
    duration_loss = sums[5] / sums[6]
    total_loss = (mel_loss + postnet_mel_loss + duration_loss
                  + pitch_loss + energy_loss)
    return (total_loss, mel_loss, postnet_mel_loss, pitch_loss,
            energy_loss, duration_loss)


# speedup vs baseline: 13.5648x; 1.8585x over previous
import jax
import jax.numpy as jnp
from jax.experimental import pallas as pl
from jax.experimental.pallas import tpu as pltpu


def _loss_body(melt_ref, melp_ref, post_ref, mmask_ref,
               pt_ref, pp_ref, et_ref, ep_ref, ldp_ref, dur_ref, tm_ref,
               total_ref, mel_ref, post_ref_o, pitch_ref, energy_ref, durl_ref,
               acc_ref):
    step = pl.program_id(0)
    nsteps = pl.num_programs(0)

    @pl.when(step == 0)
    def _():
        tm = 1.0 - tm_ref[...].astype(jnp.float32)
        pe = (pp_ref[...] - pt_ref[...]) ** 2
        ee = (ep_ref[...] - et_ref[...]) ** 2
        ldt = jnp.log(dur_ref[...].astype(jnp.float32) + 1.0)
        de = (ldp_ref[...] - ldt) ** 2
        acc_ref[0] = 0.0
        acc_ref[1] = 0.0
        acc_ref[2] = 0.0
        acc_ref[3] = jnp.sum(pe * tm)
        acc_ref[4] = jnp.sum(ee * tm)
        acc_ref[5] = jnp.sum(de * tm)
        acc_ref[6] = jnp.sum(tm)

    t = melt_ref[...]
    m = mmask_ref[...]
    mb = m[:, None, :]
    d1 = jnp.abs(melp_ref[...] - t) * mb
    d2 = jnp.abs(post_ref[...] - t) * mb
    acc_ref[0] += jnp.sum(d1)
    acc_ref[1] += jnp.sum(d2)
    acc_ref[2] += jnp.sum(m)

    @pl.when(step == nsteps - 1)
    def _():
        n_mels_f = jnp.float32(melt_ref.shape[1])
        denom = acc_ref[2] * n_mels_f
        mel_loss = acc_ref[0] / denom
        postnet_mel_loss = acc_ref[1] / denom
        tsum = acc_ref[6]
        pitch_loss = acc_ref[3] / tsum
        energy_loss = acc_ref[4] / tsum
        duration_loss = acc_ref[5] / tsum
        mel_ref[0] = mel_loss
        post_ref_o[0] = postnet_mel_loss
        pitch_ref[0] = pitch_loss
        energy_ref[0] = energy_loss
        durl_ref[0] = duration_loss
        total_ref[0] = (mel_loss + postnet_mel_loss + duration_loss
                        + pitch_loss + energy_loss)


def kernel(mel_targets, pitch_targets, energy_targets, duration_targets,
           mel_predictions, postnet_mel_predictions, pitch_predictions,
           energy_predictions, log_duration_predictions, text_masks, mel_masks):
    B, T_mel, n_mels = mel_targets.shape
    T_text = pitch_targets.shape[1]

    mt = jnp.swapaxes(mel_targets, 1, 2)
    mp = jnp.swapaxes(mel_predictions, 1, 2)
    po = jnp.swapaxes(postnet_mel_predictions, 1, 2)
    mmask = jnp.logical_not(mel_masks).astype(jnp.float32)
    tmask = text_masks.view(jnp.uint8)

    BB = 8
    scalar = jax.ShapeDtypeStruct((1,), jnp.float32)
    outs = pl.pallas_call(
        _loss_body,
        grid=(B // BB,),
        in_specs=[
            pl.BlockSpec((BB, n_mels, T_mel), lambda b: (b, 0, 0)),
            pl.BlockSpec((BB, n_mels, T_mel), lambda b: (b, 0, 0)),
            pl.BlockSpec((BB, n_mels, T_mel), lambda b: (b, 0, 0)),
            pl.BlockSpec((BB, T_mel), lambda b: (b, 0)),
            pl.BlockSpec((B, T_text), lambda b: (0, 0)),
            pl.BlockSpec((B, T_text), lambda b: (0, 0)),
            pl.BlockSpec((B, T_text), lambda b: (0, 0)),
            pl.BlockSpec((B, T_text), lambda b: (0, 0)),
            pl.BlockSpec((B, T_text), lambda b: (0, 0)),
            pl.BlockSpec((B, T_text), lambda b: (0, 0)),
            pl.BlockSpec((B, T_text), lambda b: (0, 0)),
        ],
        out_specs=[pl.BlockSpec(memory_space=pltpu.SMEM)] * 6,
        out_shape=[scalar] * 6,
        scratch_shapes=[pltpu.SMEM((8,), jnp.float32)],
    )(mt, mp, po, mmask,
      pitch_targets, pitch_predictions, energy_targets, energy_predictions,
      log_duration_predictions, duration_targets, tmask)

    total_loss, mel_loss, postnet_mel_loss, pitch_loss, energy_loss, \
        duration_loss = (o.reshape(()) for o in outs)
    return (total_loss, mel_loss, postnet_mel_loss, pitch_loss,
            energy_loss, duration_loss)


# speedup vs baseline: 13.7886x; 1.0165x over previous
import jax
import jax.numpy as jnp
from jax.experimental import pallas as pl
from jax.experimental.pallas import tpu as pltpu

_TPAD = 1024


def _loss_body(melt_ref, melp_ref, post_ref, cmask_ref, cmask_full_ref,
               pt_ref, pp_ref, et_ref, ep_ref, ldp_ref, dur_ref,
               total_ref, mel_ref, post_ref_o, pitch_ref, energy_ref, durl_ref,
               acc_ref):
    step = pl.program_id(0)
    nsteps = pl.num_programs(0)
    T_mel = melt_ref.shape[2]
    T_text = pt_ref.shape[1]

    @pl.when(step == 0)
    def _():
        tm = cmask_full_ref[:, _TPAD:_TPAD + T_text]
        pe = (pp_ref[...] - pt_ref[...]) ** 2
        ee = (ep_ref[...] - et_ref[...]) ** 2
        ldt = jnp.log(dur_ref[...].astype(jnp.float32) + 1.0)
        de = (ldp_ref[...] - ldt) ** 2
        acc_ref[0] = 0.0
        acc_ref[1] = 0.0
        acc_ref[2] = 0.0
        acc_ref[3] = jnp.sum(pe * tm)
        acc_ref[4] = jnp.sum(ee * tm)
        acc_ref[5] = jnp.sum(de * tm)
        acc_ref[6] = jnp.sum(tm)

    t = melt_ref[...]
    m = cmask_ref[:, 0:T_mel]
    mb = m[:, None, :]
    d1 = jnp.abs(melp_ref[...] - t) * mb
    d2 = jnp.abs(post_ref[...] - t) * mb
    acc_ref[0] += jnp.sum(d1)
    acc_ref[1] += jnp.sum(d2)
    acc_ref[2] += jnp.sum(m)

    @pl.when(step == nsteps - 1)
    def _():
        n_mels_f = jnp.float32(melt_ref.shape[1])
        denom = acc_ref[2] * n_mels_f
        mel_loss = acc_ref[0] / denom
        postnet_mel_loss = acc_ref[1] / denom
        tsum = acc_ref[6]
        pitch_loss = acc_ref[3] / tsum
        energy_loss = acc_ref[4] / tsum
        duration_loss = acc_ref[5] / tsum
        mel_ref[0] = mel_loss
        post_ref_o[0] = postnet_mel_loss
        pitch_ref[0] = pitch_loss
        energy_ref[0] = energy_loss
        durl_ref[0] = duration_loss
        total_ref[0] = (mel_loss + postnet_mel_loss + duration_loss
                        + pitch_loss + energy_loss)


def kernel(mel_targets, pitch_targets, energy_targets, duration_targets,
           mel_predictions, postnet_mel_predictions, pitch_predictions,
           energy_predictions, log_duration_predictions, text_masks, mel_masks):
    B, T_mel, n_mels = mel_targets.shape
    T_text = pitch_targets.shape[1]

    mt = jnp.swapaxes(mel_targets, 1, 2)
    mp = jnp.swapaxes(mel_predictions, 1, 2)
    po = jnp.swapaxes(postnet_mel_predictions, 1, 2)

    cmask = jnp.concatenate(
        [jnp.logical_not(mel_masks),
         jnp.zeros((B, _TPAD - T_mel), jnp.bool_),
         jnp.logical_not(text_masks)], axis=1).astype(jnp.float32)

    BB = 8
    W = _TPAD + T_text
    scalar = jax.ShapeDtypeStruct((1,), jnp.float32)
    outs = pl.pallas_call(
        _loss_body,
        grid=(B // BB,),
        in_specs=[
            pl.BlockSpec((BB, n_mels, T_mel), lambda b: (b, 0, 0)),
            pl.BlockSpec((BB, n_mels, T_mel), lambda b: (b, 0, 0)),
            pl.BlockSpec((BB, n_mels, T_mel), lambda b: (b, 0, 0)),
            pl.BlockSpec((BB, W), lambda b: (b, 0)),
            pl.BlockSpec((B, W), lambda b: (0, 0)),
            pl.BlockSpec((B, T_text), lambda b: (0, 0)),
            pl.BlockSpec((B, T_text), lambda b: (0, 0)),
            pl.BlockSpec((B, T_text), lambda b: (0, 0)),
            pl.BlockSpec((B, T_text), lambda b: (0, 0)),
            pl.BlockSpec((B, T_text), lambda b: (0, 0)),
            pl.BlockSpec((B, T_text), lambda b: (0, 0)),
        ],
        out_specs=[pl.BlockSpec(memory_space=pltpu.SMEM)] * 6,
        out_shape=[scalar] * 6,
        scratch_shapes=[pltpu.SMEM((8,), jnp.float32)],
    )(mt, mp, po, cmask, cmask,
      pitch_targets, pitch_predictions, energy_targets, energy_predictions,
      log_duration_predictions, duration_targets)

    total_loss, mel_loss, postnet_mel_loss, pitch_loss, energy_loss, \
        duration_loss = (o.reshape(()) for o in outs)
    return (total_loss, mel_loss, postnet_mel_loss, pitch_loss,
            energy_loss, duration_loss)
